# baseline (speedup 1.0000x reference)
import functools

import jax
import jax.numpy as jnp
from jax import lax
from jax.experimental import pallas as pl
from jax.experimental.pallas import tpu as pltpu
from jax.experimental.pallas import tpu_sc as plsc

_B, _S, _D = 4, 8192, 1024
_NC, _NS, _L = 2, 16, 16
_NW = _NC * _NS
_S_W = _S // _NW
_TB = 8
_NT = _S_W // _TB


_mesh = plsc.VectorSubcoreMesh(core_axis_name="c", subcore_axis_name="s")


@functools.partial(
    pl.kernel,
    mesh=_mesh,
    out_type=jax.ShapeDtypeStruct((_B, _S, _D), jnp.float32),
    scratch_types=[
        pltpu.VMEM((_TB, _D), jnp.float32),
        pltpu.VMEM((_TB, _D), jnp.float32),
        pltpu.VMEM((_B, _TB, _D), jnp.float32),
        pltpu.VMEM((_B, _TB, _D), jnp.float32),
        pltpu.SemaphoreType.DMA,
        pltpu.SemaphoreType.DMA,
        pltpu.SemaphoreType.DMA,
        pltpu.SemaphoreType.DMA,
        pltpu.SemaphoreType.DMA,
        pltpu.SemaphoreType.DMA,
    ],
)
def _sc_add(in_hbm, emb_hbm, out_hbm, emb0, emb1, io0, io1,
            em_s0, em_s1, ld_s0, ld_s1, st_s0, st_s1):
    wid = lax.axis_index("s") * _NC + lax.axis_index("c")
    s_base = wid * _S_W

    embs = (emb0, emb1)
    ios = (io0, io1)
    em_sems = (em_s0, em_s1)
    ld_sems = (ld_s0, ld_s1)
    st_sems = (st_s0, st_s1)

    def emb_slice(t):
        return emb_hbm.at[pl.ds(s_base + t * _TB, _TB)]

    def in_slice(t):
        return in_hbm.at[:, pl.ds(s_base + t * _TB, _TB)]

    def out_slice(t):
        return out_hbm.at[:, pl.ds(s_base + t * _TB, _TB)]

    def start_tile_loads(t, q):
        pltpu.async_copy(emb_slice(t), embs[q], em_sems[q])
        pltpu.async_copy(in_slice(t), ios[q], ld_sems[q])

    start_tile_loads(0, 0)

    def pair_body(g, _):
        for j in (0, 1):
            t = 2 * g + j
            p, q = j, 1 - j
            emb_v, io_v = embs[p], ios[p]

            pltpu.make_async_copy(emb_slice(t), emb_v, em_sems[p]).wait()
            pltpu.make_async_copy(in_slice(t), io_v, ld_sems[p]).wait()

            @pl.when(t >= 1)
            def _():
                pltpu.make_async_copy(
                    ios[q], out_slice(t - 1), st_sems[q]).wait()

            @pl.when(t < _NT - 1)
            def _():
                start_tile_loads(t + 1, q)

            def row_body(r, _):
                for c in range(_D // _L):
                    sl = pl.ds(c * _L, _L)
                    e = emb_v[r, sl]
                    for b in range(_B):
                        plsc.addupdate(io_v.at[b, r, sl], e)
                return 0

            lax.fori_loop(0, _TB, row_body, 0)

            pltpu.async_copy(io_v, out_slice(t), st_sems[p])
        return 0

    lax.fori_loop(0, _NT // 2, pair_body, 0)

    pltpu.make_async_copy(ios[1], out_slice(_NT - 1), st_sems[1]).wait()


def kernel(inputs, embeddings):
    seq_len = inputs.shape[1]
    return _sc_add(inputs, embeddings[:seq_len])


# speedup vs baseline: 1.0133x; 1.0133x over previous
import functools

import jax
import jax.numpy as jnp
from jax import lax
from jax.experimental import pallas as pl
from jax.experimental.pallas import tpu as pltpu
from jax.experimental.pallas import tpu_sc as plsc

_B, _S, _D = 4, 8192, 1024
_NC, _NS, _L = 2, 16, 16
_NW = _NC * _NS
_S_W = _S // _NW
_TB = 4
_NT = _S_W // _TB
_NR = 4
_NG = _NT // _NR
_LEAD = 2


_mesh = plsc.VectorSubcoreMesh(core_axis_name="c", subcore_axis_name="s")


@functools.partial(
    pl.kernel,
    mesh=_mesh,
    out_type=jax.ShapeDtypeStruct((_B, _S, _D), jnp.float32),
    scratch_types=[
        pltpu.VMEM((_TB, _D), jnp.float32),
        pltpu.VMEM((_TB, _D), jnp.float32),
        pltpu.VMEM((_TB, _D), jnp.float32),
        pltpu.VMEM((_TB, _D), jnp.float32),
        pltpu.VMEM((_B, _TB, _D), jnp.float32),
        pltpu.VMEM((_B, _TB, _D), jnp.float32),
        pltpu.VMEM((_B, _TB, _D), jnp.float32),
        pltpu.VMEM((_B, _TB, _D), jnp.float32),
        pltpu.SemaphoreType.DMA,
        pltpu.SemaphoreType.DMA,
        pltpu.SemaphoreType.DMA,
        pltpu.SemaphoreType.DMA,
        pltpu.SemaphoreType.DMA,
        pltpu.SemaphoreType.DMA,
        pltpu.SemaphoreType.DMA,
        pltpu.SemaphoreType.DMA,
        pltpu.SemaphoreType.DMA,
        pltpu.SemaphoreType.DMA,
        pltpu.SemaphoreType.DMA,
        pltpu.SemaphoreType.DMA,
    ],
)
def _sc_add(in_hbm, emb_hbm, out_hbm,
            e0, e1, e2, e3, o0, o1, o2, o3,
            es0, es1, es2, es3, ls0, ls1, ls2, ls3, ss0, ss1, ss2, ss3):
    wid = lax.axis_index("s") * _NC + lax.axis_index("c")
    s_base = wid * _S_W

    embs = (e0, e1, e2, e3)
    ios = (o0, o1, o2, o3)
    em_sems = (es0, es1, es2, es3)
    ld_sems = (ls0, ls1, ls2, ls3)
    st_sems = (ss0, ss1, ss2, ss3)

    def emb_slice(t):
        return emb_hbm.at[pl.ds(s_base + t * _TB, _TB)]

    def in_slice(t):
        return in_hbm.at[:, pl.ds(s_base + t * _TB, _TB)]

    def out_slice(t):
        return out_hbm.at[:, pl.ds(s_base + t * _TB, _TB)]

    def start_tile_loads(t, p):
        pltpu.async_copy(emb_slice(t), embs[p], em_sems[p])
        pltpu.async_copy(in_slice(t), ios[p], ld_sems[p])

    def wait_store(t, p):
        pltpu.make_async_copy(ios[p], out_slice(t), st_sems[p]).wait()

    start_tile_loads(0, 0)
    start_tile_loads(1, 1)

    def group_body(g, _):
        for j in range(_NR):
            t = g * _NR + j
            p = j
            pu = (j + _LEAD) % _NR
            u = t + _LEAD

            if j < _LEAD:
                @pl.when(g == 0)
                def _():
                    start_tile_loads(u, pu)

                @pl.when(g >= 1)
                def _():
                    wait_store(u - _NR, pu)
                    start_tile_loads(u, pu)
            else:
                @pl.when(g < _NG - 1)
                def _():
                    wait_store(u - _NR, pu)
                    start_tile_loads(u, pu)

            pltpu.make_async_copy(emb_slice(t), embs[p], em_sems[p]).wait()
            pltpu.make_async_copy(in_slice(t), ios[p], ld_sems[p]).wait()

            emb_v, io_v = embs[p], ios[p]

            def row_body(r, _):
                for c in range(_D // _L):
                    sl = pl.ds(c * _L, _L)
                    e = emb_v[r, sl]
                    for b in range(_B):
                        plsc.addupdate(io_v.at[b, r, sl], e)
                return 0

            lax.fori_loop(0, _TB, row_body, 0)

            pltpu.async_copy(ios[p], out_slice(t), st_sems[p])
        return 0

    lax.fori_loop(0, _NG, group_body, 0)

    for p in range(_NR):
        wait_store(_NT - _NR + p, p)


def kernel(inputs, embeddings):
    seq_len = inputs.shape[1]
    return _sc_add(inputs, embeddings[:seq_len])


# speedup vs baseline: 1.0293x; 1.0159x over previous
import functools

import jax
import jax.numpy as jnp
from jax import lax
from jax.experimental import pallas as pl
from jax.experimental.pallas import tpu as pltpu
from jax.experimental.pallas import tpu_sc as plsc

_B, _S, _D = 4, 8192, 1024
_NC, _NS, _L = 2, 16, 16
_NW = _NC * _NS
_S_W = _S // _NW
_TB = 4
_NT = _S_W // _TB
_NR = 4
_NG = _NT // _NR
_LEAD = 2


_mesh = plsc.VectorSubcoreMesh(core_axis_name="c", subcore_axis_name="s")


@functools.partial(
    pl.kernel,
    mesh=_mesh,
    out_type=jax.ShapeDtypeStruct((_B, _S, _D), jnp.float32),
    scratch_types=[
        pltpu.VMEM((_TB, _D), jnp.float32),
        pltpu.VMEM((_TB, _D), jnp.float32),
        pltpu.VMEM((_TB, _D), jnp.float32),
        pltpu.VMEM((_TB, _D), jnp.float32),
        pltpu.VMEM((_B, _TB, _D), jnp.float32),
        pltpu.VMEM((_B, _TB, _D), jnp.float32),
        pltpu.VMEM((_B, _TB, _D), jnp.float32),
        pltpu.VMEM((_B, _TB, _D), jnp.float32),
        pltpu.SemaphoreType.DMA,
        pltpu.SemaphoreType.DMA,
        pltpu.SemaphoreType.DMA,
        pltpu.SemaphoreType.DMA,
        pltpu.SemaphoreType.DMA,
        pltpu.SemaphoreType.DMA,
        pltpu.SemaphoreType.DMA,
        pltpu.SemaphoreType.DMA,
        pltpu.SemaphoreType.DMA,
        pltpu.SemaphoreType.DMA,
        pltpu.SemaphoreType.DMA,
        pltpu.SemaphoreType.DMA,
    ],
)
def _sc_add(in_hbm, emb_hbm, out_hbm,
            e0, e1, e2, e3, o0, o1, o2, o3,
            es0, es1, es2, es3, ls0, ls1, ls2, ls3, ss0, ss1, ss2, ss3):
    wid = lax.axis_index("s") * _NC + lax.axis_index("c")
    s_base = wid * _S_W

    embs = (e0, e1, e2, e3)
    ios = (o0, o1, o2, o3)
    em_sems = (es0, es1, es2, es3)
    ld_sems = (ls0, ls1, ls2, ls3)
    st_sems = (ss0, ss1, ss2, ss3)

    def emb_slice(t):
        return emb_hbm.at[pl.ds(s_base + t * _TB, _TB)]

    def in_slice(t):
        return in_hbm.at[:, pl.ds(s_base + t * _TB, _TB)]

    def out_slice(t):
        return out_hbm.at[:, pl.ds(s_base + t * _TB, _TB)]

    def start_tile_loads(t, p):
        pltpu.async_copy(emb_slice(t), embs[p], em_sems[p])
        pltpu.async_copy(in_slice(t), ios[p], ld_sems[p])

    def wait_store(t, p):
        pltpu.make_async_copy(ios[p], out_slice(t), st_sems[p]).wait()

    start_tile_loads(0, 0)
    start_tile_loads(1, 1)

    def group_body(g, _):
        for j in range(_NR):
            t = g * _NR + j
            p = j
            pu = (j + _LEAD) % _NR
            u = t + _LEAD

            if j < _LEAD:
                @pl.when(g == 0)
                def _():
                    start_tile_loads(u, pu)

                @pl.when(g >= 1)
                def _():
                    wait_store(u - _NR, pu)
                    start_tile_loads(u, pu)
            else:
                @pl.when(g < _NG - 1)
                def _():
                    wait_store(u - _NR, pu)
                    start_tile_loads(u, pu)

            pltpu.make_async_copy(emb_slice(t), embs[p], em_sems[p]).wait()
            pltpu.make_async_copy(in_slice(t), ios[p], ld_sems[p]).wait()

            emb_v, io_v = embs[p], ios[p]

            def row_body(r, _):
                for c in range(_D // _L):
                    sl = pl.ds(c * _L, _L)
                    e = emb_v[r, sl]
                    for b in range(_B):
                        io_v[b, r, sl] = io_v[b, r, sl] + e
                return 0

            lax.fori_loop(0, _TB, row_body, 0)

            pltpu.async_copy(ios[p], out_slice(t), st_sems[p])
        return 0

    lax.fori_loop(0, _NG, group_body, 0)

    for p in range(_NR):
        wait_store(_NT - _NR + p, p)


def kernel(inputs, embeddings):
    seq_len = inputs.shape[1]
    return _sc_add(inputs, embeddings[:seq_len])
